# baseline (speedup 1.0000x reference)
import functools

import jax
import jax.numpy as jnp
from jax import lax
from jax.experimental import pallas as pl
from jax.experimental.pallas import tpu as pltpu
from jax.experimental.pallas import tpu_sc as plsc

_N = 10000
_E = 320000
_DIMS = [128] + [256] * 7 + [128] * 7 + [64] * 7 + [32] * 7 + [16] * 7 + [8] * 7 + [4] * 6

_NP = 10240
_NC, _NS = 2, 16
_CH = 125
_NROW = _E // _CH
_RPT = _NP // _NS
_ZR = 32
_UNR = 10
_NBUF = 5
_NG = 5

_MB = 2000


def _make_spmm(Fh, mode):
    nit = _NROW // _NS if mode == "feat" else _NROW // (_NC * _NS)
    nblk = nit // _UNR
    z_shape = (2, _N, Fh) if mode == "feat" else (_N, Fh)
    mesh = plsc.VectorSubcoreMesh(
        core_axis_name="c", subcore_axis_name="s", num_cores=_NC, num_subcores=_NS
    )

    @functools.partial(
        pl.kernel,
        out_type=jax.ShapeDtypeStruct((2, _NP, Fh), jnp.float32),
        mesh=mesh,
        scratch_types=[
            pltpu.VMEM((2 * _UNR, 2, _CH), jnp.int32),
            pltpu.VMEM((_NBUF, _CH, Fh), jnp.float32),
            pltpu.VMEM((_ZR, Fh), jnp.float32),
            pltpu.VMEM_SHARED((_NP, Fh), jnp.float32),
        ]
        + [pltpu.SemaphoreType.DMA] * (2 * _NBUF + 2),
        compiler_params=pltpu.CompilerParams(use_tc_tiling_on_sc=False),
        name=f"sage_spmm_{mode}{Fh}",
    )
    def spmm(z_hbm, ei_hbm, out_hbm, idx_v, rows_v, zero_v, acc_sh, *sems):
        gsems = sems[:_NBUF]
        ssems = sems[_NBUF:2 * _NBUF]
        semi = sems[2 * _NBUF]
        semz = sems[2 * _NBUF + 1]
        c = lax.axis_index("c")
        s = lax.axis_index("s")
        if mode == "feat":
            base = s * nit
            zsrc = z_hbm.at[c]
        else:
            base = (s * _NC + c) * nit
            zsrc = z_hbm
        for a in range(_ZR):
            for k in range(Fh // 16):
                zero_v[a, pl.ds(k * 16, 16)] = jnp.zeros((16,), jnp.float32)
        row0 = s * _RPT
        for a in range(_RPT // _ZR):
            pltpu.sync_copy(zero_v, acc_sh.at[pl.ds(row0 + a * _ZR, _ZR)])
        plsc.subcore_barrier()
        pltpu.async_copy(
            ei_hbm.at[pl.ds(base, _UNR)], idx_v.at[pl.ds(0, _UNR)], semi
        )

        def gather(row, buf):
            return pltpu.async_copy(
                zsrc.at[idx_v.at[row, 0]], rows_v.at[buf], gsems[buf]
            )

        def body(o, carry):
            p = lax.rem(o, 2)
            q = 1 - p
            ioff = p * _UNR
            pltpu.make_async_copy(
                ei_hbm.at[pl.ds(base, _UNR)], idx_v.at[pl.ds(0, _UNR)], semi
            ).wait()

            @pl.when(o + 1 < nblk)
            def _():
                pltpu.async_copy(
                    ei_hbm.at[pl.ds(base + (o + 1) * _UNR, _UNR)],
                    idx_v.at[pl.ds(q * _UNR, _UNR)],
                    semi,
                )

            descs = [None] * _UNR
            for k in range(_NG):
                descs[k] = gather(ioff + k, k)
            for k in range(_UNR):
                descs[k].wait()
                pltpu.sync_copy(
                    rows_v.at[k % _NBUF], acc_sh.at[idx_v.at[ioff + k, 1]], add=True
                )
                j = k + _NG
                if j < _UNR:
                    descs[j] = gather(ioff + j, j % _NBUF)
            return carry

        lax.fori_loop(0, nblk, body, 0)
        plsc.subcore_barrier()
        pltpu.sync_copy(
            acc_sh.at[pl.ds(row0, _RPT)], out_hbm.at[c, pl.ds(row0, _RPT)]
        )

    return spmm


_SPMM_CACHE = {}


def _spmm(z, ei3):
    if z.ndim == 3:
        key = ("feat", z.shape[2])
    else:
        key = ("edge", z.shape[1])
    if key not in _SPMM_CACHE:
        _SPMM_CACHE[key] = _make_spmm(key[1], key[0])
    return _SPMM_CACHE[key](z, ei3)


def _dinv_block(degp_blk):
    deg = degp_blk[0, :, 0:1] + degp_blk[1, :, 0:1]
    return jnp.where(deg > 0.0, 1.0 / jnp.maximum(deg, 1.0), 0.0)


def _row_spec(width):
    return pl.BlockSpec((_MB, width), lambda m: (m, 0))


def _part_spec(width):
    return pl.BlockSpec((2, _MB, width), lambda m: (0, m, 0))


def _full_spec(*shape):
    return pl.BlockSpec(shape, lambda m: (0,) * len(shape))


def _h_layout(f):
    if f >= 32:
        return ("feat", 2 if f == 256 else 1, min(f, 128) // 2)
    return ("edge", 1, max(16, f))


def _assemble_h(mi, fi, refs):
    if mi == "feat":
        return jnp.concatenate(
            [piece for r in refs for piece in (r[0], r[1])], axis=1
        )
    return refs[0][...][:, :fi]


def _make_tc_r(fi, fo):
    mi, ni, wi = _h_layout(fi)

    def body(*refs):
        h_refs = refs[:ni]
        wr_ref = refs[ni]
        o_ref = refs[ni + 1]
        h = _assemble_h(mi, fi, h_refs)
        o_ref[...] = jnp.dot(h, wr_ref[...], preferred_element_type=jnp.float32)

    in_specs = (
        ([_part_spec(wi)] * ni if mi == "feat" else [_row_spec(wi)])
        + [_full_spec(fi, fo)]
    )

    def run(hs, Wr):
        return pl.pallas_call(
            body,
            grid=(_N // _MB,),
            in_specs=in_specs,
            out_specs=_row_spec(fo),
            out_shape=jax.ShapeDtypeStruct((_N, fo), jnp.float32),
        )(*(list(hs) + [Wr]))

    return run


def _make_tc_agg(fi, fo, last):
    mi, ni, wi = _h_layout(fi)
    if last:
        mo, no, wo = ("edge", 1, 1)
    else:
        mo, no, wo = _h_layout(fo)

    def body(*refs):
        sp_refs = refs[:ni]
        r_ref, degp_ref, b_ref, wl_ref = refs[ni:ni + 4]
        pos = ni + 4
        if last:
            wreg_ref, breg_ref = refs[pos:pos + 2]
            pos += 2
        out_refs = refs[pos:]

        dinv = _dinv_block(degp_ref[...])
        if mi == "feat":
            s = _assemble_h(mi, fi, sp_refs)
        else:
            s = (sp_refs[0][0] + sp_refs[0][1])[:, :fi]
        hn = jnp.maximum(
            jnp.dot(s * dinv, wl_ref[...], preferred_element_type=jnp.float32)
            + r_ref[...] + b_ref[...],
            0.0,
        )
        if last:
            out_refs[0][...] = (
                jnp.dot(hn, wreg_ref[...], preferred_element_type=jnp.float32)
                + breg_ref[...]
            )
        elif mo == "feat":
            for a, o_ref in enumerate(out_refs):
                o_ref[0] = hn[:, 2 * a * wo:(2 * a + 1) * wo]
                o_ref[1] = hn[:, (2 * a + 1) * wo:(2 * a + 2) * wo]
        else:
            if fo < 16:
                hn = jnp.concatenate(
                    [hn, jnp.zeros((hn.shape[0], 16 - fo), jnp.float32)], axis=1
                )
            out_refs[0][...] = hn

    in_specs = (
        [_part_spec(wi)] * ni
        + [_row_spec(fo), _part_spec(16), _full_spec(1, fo), _full_spec(fi, fo)]
    )
    if last:
        in_specs += [_full_spec(fo, 1), _full_spec(1, 1)]
        out_specs = [_row_spec(1)]
        out_shape = [jax.ShapeDtypeStruct((_N, 1), jnp.float32)]
    elif mo == "feat":
        out_specs = [_part_spec(wo)] * no
        out_shape = [jax.ShapeDtypeStruct((2, _N, wo), jnp.float32)] * no
    else:
        out_specs = [_row_spec(wo)]
        out_shape = [jax.ShapeDtypeStruct((_N, wo), jnp.float32)]

    def run(sps, r, degp, b, Wl, head=None):
        args = list(sps) + [r, degp, b, Wl]
        if last:
            args += [head[0], head[1]]
        return pl.pallas_call(
            body,
            grid=(_N // _MB,),
            in_specs=in_specs,
            out_specs=out_specs,
            out_shape=out_shape,
        )(*args)

    return run


def _split_x(x):
    def body(x_ref, o_ref):
        o_ref[0] = x_ref[:, :64]
        o_ref[1] = x_ref[:, 64:]

    return pl.pallas_call(
        body,
        grid=(_N // _MB,),
        in_specs=[_row_spec(128)],
        out_specs=_part_spec(64),
        out_shape=jax.ShapeDtypeStruct((2, _N, 64), jnp.float32),
    )(x)


def kernel(x, edge_index, params):
    ei3 = jnp.stack(
        [edge_index[0].reshape(_NROW, _CH), edge_index[1].reshape(_NROW, _CH)],
        axis=1,
    )

    ones16 = jnp.ones((_N, 16), jnp.float32)
    degp = _spmm(ones16, ei3)

    hs = [_split_x(x)]
    r = _make_tc_r(128, 256)(hs, params["Wr_0"])
    for i in range(48):
        fi, fo = _DIMS[i], _DIMS[i + 1]
        last = i == 47
        sps = [_spmm(h, ei3) for h in hs]
        agg = _make_tc_agg(fi, fo, last)
        b = params[f"b_{i}"].reshape(1, -1)
        if last:
            return agg(
                sps, r, degp, b, params[f"Wl_{i}"],
                head=(params["W_reg"], params["b_reg"].reshape(1, 1)),
            )[0]
        hs = list(agg(sps, r, degp, b, params[f"Wl_{i}"]))
        r = _make_tc_r(fo, _DIMS[i + 2])(hs, params[f"Wr_{i + 1}"])


# speedup vs baseline: 1.0077x; 1.0077x over previous
import functools

import jax
import jax.numpy as jnp
from jax import lax
from jax.experimental import pallas as pl
from jax.experimental.pallas import tpu as pltpu
from jax.experimental.pallas import tpu_sc as plsc

_N = 10000
_E = 320000
_DIMS = [128] + [256] * 7 + [128] * 7 + [64] * 7 + [32] * 7 + [16] * 7 + [8] * 7 + [4] * 6

_NP = 10240
_NC, _NS = 2, 16
_CH = 125
_NROW = _E // _CH
_RPT = _NP // _NS
_ZR = 32
_UNR = 10
_NBUF = 5
_NG = 5

_MB = 2000


def _make_spmm(Fh, mode):
    nit = _NROW // _NS if mode == "feat" else _NROW // (_NC * _NS)
    nblk = nit // _UNR
    z_shape = (2, _N, Fh) if mode == "feat" else (_N, Fh)
    mesh = plsc.VectorSubcoreMesh(
        core_axis_name="c", subcore_axis_name="s", num_cores=_NC, num_subcores=_NS
    )

    @functools.partial(
        pl.kernel,
        out_type=jax.ShapeDtypeStruct((2, _NP, Fh), jnp.float32),
        mesh=mesh,
        scratch_types=[
            pltpu.VMEM((2 * _UNR, 2, _CH), jnp.int32),
            pltpu.VMEM((_NBUF, _CH, Fh), jnp.float32),
            pltpu.VMEM((_ZR, Fh), jnp.float32),
            pltpu.VMEM_SHARED((_NP, Fh), jnp.float32),
        ]
        + [pltpu.SemaphoreType.DMA] * (2 * _NBUF + 2),
        compiler_params=pltpu.CompilerParams(use_tc_tiling_on_sc=False),
        name=f"sage_spmm_{mode}{Fh}",
    )
    def spmm(z_hbm, ei_hbm, out_hbm, idx_v, rows_v, zero_v, acc_sh, *sems):
        gsems = sems[:_NBUF]
        ssems = sems[_NBUF:2 * _NBUF]
        semi = sems[2 * _NBUF]
        semz = sems[2 * _NBUF + 1]
        c = lax.axis_index("c")
        s = lax.axis_index("s")
        if mode == "feat":
            base = s * nit
            zsrc = z_hbm.at[c]
        else:
            base = (s * _NC + c) * nit
            zsrc = z_hbm
        for a in range(_ZR):
            for k in range(Fh // 16):
                zero_v[a, pl.ds(k * 16, 16)] = jnp.zeros((16,), jnp.float32)
        row0 = s * _RPT
        zdescs = [
            pltpu.async_copy(zero_v, acc_sh.at[pl.ds(row0 + a * _ZR, _ZR)], semz)
            for a in range(_RPT // _ZR)
        ]
        for d in zdescs:
            d.wait()
        plsc.subcore_barrier()
        pltpu.async_copy(
            ei_hbm.at[pl.ds(base, _UNR)], idx_v.at[pl.ds(0, _UNR)], semi
        )

        def gather(row, buf):
            return pltpu.async_copy(
                zsrc.at[idx_v.at[row, 0]], rows_v.at[buf], gsems[buf]
            )

        def body(o, carry):
            p = lax.rem(o, 2)
            q = 1 - p
            ioff = p * _UNR
            pltpu.make_async_copy(
                ei_hbm.at[pl.ds(base, _UNR)], idx_v.at[pl.ds(0, _UNR)], semi
            ).wait()

            @pl.when(o + 1 < nblk)
            def _():
                pltpu.async_copy(
                    ei_hbm.at[pl.ds(base + (o + 1) * _UNR, _UNR)],
                    idx_v.at[pl.ds(q * _UNR, _UNR)],
                    semi,
                )

            descs = [None] * _UNR
            for k in range(_NG):
                descs[k] = gather(ioff + k, k)
            for k in range(_UNR):
                descs[k].wait()
                pltpu.sync_copy(
                    rows_v.at[k % _NBUF], acc_sh.at[idx_v.at[ioff + k, 1]], add=True
                )
                j = k + _NG
                if j < _UNR:
                    descs[j] = gather(ioff + j, j % _NBUF)
            return carry

        lax.fori_loop(0, nblk, body, 0)
        plsc.subcore_barrier()
        pltpu.sync_copy(
            acc_sh.at[pl.ds(row0, _RPT)], out_hbm.at[c, pl.ds(row0, _RPT)]
        )

    return spmm


_SPMM_CACHE = {}


def _spmm(z, ei3):
    if z.ndim == 3:
        key = ("feat", z.shape[2])
    else:
        key = ("edge", z.shape[1])
    if key not in _SPMM_CACHE:
        _SPMM_CACHE[key] = _make_spmm(key[1], key[0])
    return _SPMM_CACHE[key](z, ei3)


def _dinv_block(degp_blk):
    deg = degp_blk[0, :, 0:1] + degp_blk[1, :, 0:1]
    return jnp.where(deg > 0.0, 1.0 / jnp.maximum(deg, 1.0), 0.0)


def _row_spec(width):
    return pl.BlockSpec((_MB, width), lambda m: (m, 0))


def _part_spec(width):
    return pl.BlockSpec((2, _MB, width), lambda m: (0, m, 0))


def _full_spec(*shape):
    return pl.BlockSpec(shape, lambda m: (0,) * len(shape))


def _h_layout(f):
    if f >= 32:
        return ("feat", 2 if f == 256 else 1, min(f, 128) // 2)
    return ("edge", 1, max(16, f))


def _assemble_h(mi, fi, refs):
    if mi == "feat":
        return jnp.concatenate(
            [piece for r in refs for piece in (r[0], r[1])], axis=1
        )
    return refs[0][...][:, :fi]


def _make_tc_r(fi, fo):
    mi, ni, wi = _h_layout(fi)

    def body(*refs):
        h_refs = refs[:ni]
        wr_ref = refs[ni]
        o_ref = refs[ni + 1]
        h = _assemble_h(mi, fi, h_refs)
        o_ref[...] = jnp.dot(h, wr_ref[...], preferred_element_type=jnp.float32)

    in_specs = (
        ([_part_spec(wi)] * ni if mi == "feat" else [_row_spec(wi)])
        + [_full_spec(fi, fo)]
    )

    def run(hs, Wr):
        return pl.pallas_call(
            body,
            grid=(_N // _MB,),
            in_specs=in_specs,
            out_specs=_row_spec(fo),
            out_shape=jax.ShapeDtypeStruct((_N, fo), jnp.float32),
        )(*(list(hs) + [Wr]))

    return run


def _make_tc_agg(fi, fo, last):
    mi, ni, wi = _h_layout(fi)
    if last:
        mo, no, wo = ("edge", 1, 1)
    else:
        mo, no, wo = _h_layout(fo)

    def body(*refs):
        sp_refs = refs[:ni]
        r_ref, degp_ref, b_ref, wl_ref = refs[ni:ni + 4]
        pos = ni + 4
        if last:
            wreg_ref, breg_ref = refs[pos:pos + 2]
            pos += 2
        out_refs = refs[pos:]

        dinv = _dinv_block(degp_ref[...])
        if mi == "feat":
            s = _assemble_h(mi, fi, sp_refs)
        else:
            s = (sp_refs[0][0] + sp_refs[0][1])[:, :fi]
        hn = jnp.maximum(
            jnp.dot(s * dinv, wl_ref[...], preferred_element_type=jnp.float32)
            + r_ref[...] + b_ref[...],
            0.0,
        )
        if last:
            out_refs[0][...] = (
                jnp.dot(hn, wreg_ref[...], preferred_element_type=jnp.float32)
                + breg_ref[...]
            )
        elif mo == "feat":
            for a, o_ref in enumerate(out_refs):
                o_ref[0] = hn[:, 2 * a * wo:(2 * a + 1) * wo]
                o_ref[1] = hn[:, (2 * a + 1) * wo:(2 * a + 2) * wo]
        else:
            if fo < 16:
                hn = jnp.concatenate(
                    [hn, jnp.zeros((hn.shape[0], 16 - fo), jnp.float32)], axis=1
                )
            out_refs[0][...] = hn

    in_specs = (
        [_part_spec(wi)] * ni
        + [_row_spec(fo), _part_spec(16), _full_spec(1, fo), _full_spec(fi, fo)]
    )
    if last:
        in_specs += [_full_spec(fo, 1), _full_spec(1, 1)]
        out_specs = [_row_spec(1)]
        out_shape = [jax.ShapeDtypeStruct((_N, 1), jnp.float32)]
    elif mo == "feat":
        out_specs = [_part_spec(wo)] * no
        out_shape = [jax.ShapeDtypeStruct((2, _N, wo), jnp.float32)] * no
    else:
        out_specs = [_row_spec(wo)]
        out_shape = [jax.ShapeDtypeStruct((_N, wo), jnp.float32)]

    def run(sps, r, degp, b, Wl, head=None):
        args = list(sps) + [r, degp, b, Wl]
        if last:
            args += [head[0], head[1]]
        return pl.pallas_call(
            body,
            grid=(_N // _MB,),
            in_specs=in_specs,
            out_specs=out_specs,
            out_shape=out_shape,
        )(*args)

    return run


def _split_x(x):
    def body(x_ref, o_ref):
        o_ref[0] = x_ref[:, :64]
        o_ref[1] = x_ref[:, 64:]

    return pl.pallas_call(
        body,
        grid=(_N // _MB,),
        in_specs=[_row_spec(128)],
        out_specs=_part_spec(64),
        out_shape=jax.ShapeDtypeStruct((2, _N, 64), jnp.float32),
    )(x)


def kernel(x, edge_index, params):
    ei3 = jnp.stack(
        [edge_index[0].reshape(_NROW, _CH), edge_index[1].reshape(_NROW, _CH)],
        axis=1,
    )

    ones16 = jnp.ones((_N, 16), jnp.float32)
    degp = _spmm(ones16, ei3)

    hs = [_split_x(x)]
    r = _make_tc_r(128, 256)(hs, params["Wr_0"])
    for i in range(48):
        fi, fo = _DIMS[i], _DIMS[i + 1]
        last = i == 47
        sps = [_spmm(h, ei3) for h in hs]
        agg = _make_tc_agg(fi, fo, last)
        b = params[f"b_{i}"].reshape(1, -1)
        if last:
            return agg(
                sps, r, degp, b, params[f"Wl_{i}"],
                head=(params["W_reg"], params["b_reg"].reshape(1, 1)),
            )[0]
        hs = list(agg(sps, r, degp, b, params[f"Wl_{i}"]))
        r = _make_tc_r(fo, _DIMS[i + 2])(hs, params[f"Wr_{i + 1}"])
